# baseline (speedup 1.0000x reference)
import functools

import jax
import jax.numpy as jnp
from jax.experimental import pallas as pl
from jax.experimental.pallas import tpu as pltpu


def _taps(parity):
    return ((0, 1),) if parity == 0 else ((1, 0), (0, 2))


def _phase_weights(dw2, CMID, COUT):
    dwc = jnp.zeros((8, CMID, 8 * COUT), dw2.dtype)
    for od in range(2):
        for oh in range(2):
            for ow in range(2):
                s = od * 4 + oh * 2 + ow
                for pd in range(2):
                    for ph in range(2):
                        for pw in range(2):
                            p = pd * 4 + ph * 2 + pw
                            dims = []
                            ok = True
                            for parity, off in ((pd, od), (ph, oh), (pw, ow)):
                                k = dict(_taps(parity)).get(off)
                                if k is None:
                                    ok = False
                                    break
                                dims.append(k)
                            if ok:
                                kd, kh, kw = dims
                                dwc = dwc.at[s, :, p * COUT:(p + 1) * COUT
                                             ].set(dw2[kd, kh, kw])
    return dwc


def _fused_kernel(x_ref, w_ref, b_ref, dwc_ref, db_ref, o_ref,
                  h_ref, xs_ref, hs_ref, *, D, H, W, CMID, COUT):
    cin = x_ref.shape[-1]
    R = H * W
    q = pl.program_id(1)

    @pl.when(q == 0)
    def _conv_step():
        for b in range(3):
            for c in range(3):
                xs_ref[b, c] = x_ref[0, :, b:b + H, c:c + W, :].reshape(
                    (D + 2) * R, cin)

        h_ref[...] = jnp.zeros_like(h_ref)
        DC = 8 if D % 8 == 0 else D
        for d0 in range(0, D, DC):
            M = DC * R
            acc = jnp.zeros((M, CMID), jnp.float32)
            for a in range(3):
                for b in range(3):
                    for c in range(3):
                        tap = xs_ref[b, c, (d0 + a) * R:(d0 + a + DC) * R, :]
                        acc = acc + jnp.dot(
                            tap, w_ref[a, b, c],
                            preferred_element_type=jnp.float32)
            acc = jnp.maximum(acc + b_ref[...], 0.0)
            h_ref[d0:d0 + DC, :H, :W, :] = acc.reshape(
                DC, H, W, CMID).astype(jnp.bfloat16)

        for oh in range(2):
            for ow in range(2):
                hs_ref[oh, ow] = h_ref[:, oh:oh + H, ow:ow + W, :].reshape(
                    (D + 1) * R, CMID)

    DQ = D // 2
    DCH = 2
    for mc in range(DQ // DCH):
        M = DCH * R
        acc = jnp.zeros((M, 8 * COUT), jnp.float32)
        for od in range(2):
            for oh in range(2):
                for ow in range(2):
                    s = od * 4 + oh * 2 + ow
                    tap = hs_ref[oh, ow,
                                 pl.ds((q * DQ + mc * DCH + od) * R, M), :]
                    acc = acc + jnp.dot(
                        tap, dwc_ref[s],
                        preferred_element_type=jnp.float32)
        acc = jnp.maximum(acc + db_ref[...], 0.0)
        o_ref[0, mc * DCH:(mc + 1) * DCH] = acc.reshape(
            DCH, H, W, 8 * COUT)


def kernel(conv_w, conv_b, deconv_w, deconv_b, x_ncdhw):
    N, CIN, D, H, W = x_ncdhw.shape
    CMID = conv_w.shape[0]
    COUT = deconv_w.shape[1]

    x = jnp.transpose(x_ncdhw, (0, 2, 3, 4, 1))
    xp = jnp.pad(x, ((0, 0), (1, 1), (1, 1), (1, 1), (0, 0))).astype(
        jnp.bfloat16)
    w2 = jnp.transpose(conv_w, (2, 3, 4, 1, 0)).astype(jnp.bfloat16)
    dw2 = jnp.transpose(deconv_w, (2, 3, 4, 0, 1)).astype(jnp.bfloat16)
    dwc = _phase_weights(dw2, CMID, COUT)
    b2 = conv_b.reshape(1, CMID).astype(jnp.float32)
    db2 = jnp.tile(deconv_b.reshape(1, COUT), (1, 8)).astype(jnp.float32)

    body = functools.partial(_fused_kernel, D=D, H=H, W=W, CMID=CMID,
                             COUT=COUT)
    yph = pl.pallas_call(
        body,
        out_shape=jax.ShapeDtypeStruct((N, D, H, W, 8 * COUT), jnp.float32),
        grid=(N, 2),
        in_specs=[
            pl.BlockSpec((1, D + 2, H + 2, W + 2, CIN),
                         lambda n, q: (n, 0, 0, 0, 0)),
            pl.BlockSpec((3, 3, 3, CIN, CMID), lambda n, q: (0, 0, 0, 0, 0)),
            pl.BlockSpec((1, CMID), lambda n, q: (0, 0)),
            pl.BlockSpec((8, CMID, 8 * COUT), lambda n, q: (0, 0, 0)),
            pl.BlockSpec((1, 8 * COUT), lambda n, q: (0, 0)),
        ],
        out_specs=pl.BlockSpec((1, D // 2, H, W, 8 * COUT),
                               lambda n, q: (n, q, 0, 0, 0)),
        scratch_shapes=[
            pltpu.VMEM((D + 1, H + 1, W + 1, CMID), jnp.bfloat16),
            pltpu.VMEM((3, 3, (D + 2) * H * W, CIN), jnp.bfloat16),
            pltpu.VMEM((2, 2, (D + 1) * H * W, CMID), jnp.bfloat16),
        ],
        compiler_params=pltpu.CompilerParams(
            dimension_semantics=("parallel", "arbitrary")),
    )(xp, w2, b2, dwc, db2)

    yph = yph.reshape(N, D, H, W, 2, 2, 2, COUT)
    y = jnp.transpose(yph, (0, 7, 1, 4, 2, 5, 3, 6))
    return y.reshape(N, COUT, 2 * D, 2 * H, 2 * W)


# speedup vs baseline: 1.0137x; 1.0137x over previous
import functools

import jax
import jax.numpy as jnp
from jax.experimental import pallas as pl
from jax.experimental.pallas import tpu as pltpu


def _taps(parity):
    return ((0, 1),) if parity == 0 else ((1, 0), (0, 2))


def _phase_weights(dw2, CMID, COUT):
    dwc = jnp.zeros((8, CMID, 8 * COUT), dw2.dtype)
    for od in range(2):
        for oh in range(2):
            for ow in range(2):
                s = od * 4 + oh * 2 + ow
                for pd in range(2):
                    for ph in range(2):
                        for pw in range(2):
                            p = pd * 4 + ph * 2 + pw
                            dims = []
                            ok = True
                            for parity, off in ((pd, od), (ph, oh), (pw, ow)):
                                k = dict(_taps(parity)).get(off)
                                if k is None:
                                    ok = False
                                    break
                                dims.append(k)
                            if ok:
                                kd, kh, kw = dims
                                dwc = dwc.at[s, :, p * COUT:(p + 1) * COUT
                                             ].set(dw2[kd, kh, kw])
    return dwc


def _fused_kernel(x_ref, w_ref, b_ref, dwc_ref, db_ref, o_ref,
                  h_ref, xs_ref, hs_ref, *, D, H, W, CMID, COUT):
    cin = x_ref.shape[-1]
    R = H * W
    if True:
        for b in range(3):
            for c in range(3):
                xs_ref[b, c] = x_ref[0, :, b:b + H, c:c + W, :].reshape(
                    (D + 2) * R, cin)

        h_ref[...] = jnp.zeros_like(h_ref)
        DC = 8 if D % 8 == 0 else D
        for d0 in range(0, D, DC):
            M = DC * R
            acc = jnp.zeros((M, CMID), jnp.float32)
            for a in range(3):
                for b in range(3):
                    for c in range(3):
                        tap = xs_ref[b, c, (d0 + a) * R:(d0 + a + DC) * R, :]
                        acc = acc + jnp.dot(
                            tap, w_ref[a, b, c],
                            preferred_element_type=jnp.float32)
            acc = jnp.maximum(acc + b_ref[...], 0.0)
            h_ref[d0:d0 + DC, :H, :W, :] = acc.reshape(
                DC, H, W, CMID).astype(jnp.bfloat16)

        for oh in range(2):
            for ow in range(2):
                hs_ref[oh, ow] = h_ref[:, oh:oh + H, ow:ow + W, :].reshape(
                    (D + 1) * R, CMID)

    DCH = 2
    for mc in range(D // DCH):
        M = DCH * R
        acc = jnp.zeros((M, 8 * COUT), jnp.float32)
        for od in range(2):
            for oh in range(2):
                for ow in range(2):
                    s = od * 4 + oh * 2 + ow
                    d0 = (mc * DCH + od) * R
                    tap = hs_ref[oh, ow, d0:d0 + M, :]
                    acc = acc + jnp.dot(
                        tap, dwc_ref[s],
                        preferred_element_type=jnp.float32)
        acc = jnp.maximum(acc + db_ref[...], 0.0)
        o_ref[0, mc * DCH:(mc + 1) * DCH] = acc.reshape(
            DCH, H, W, 8 * COUT)


def kernel(conv_w, conv_b, deconv_w, deconv_b, x_ncdhw):
    N, CIN, D, H, W = x_ncdhw.shape
    CMID = conv_w.shape[0]
    COUT = deconv_w.shape[1]

    x = jnp.transpose(x_ncdhw, (0, 2, 3, 4, 1))
    xp = jnp.pad(x, ((0, 0), (1, 1), (1, 1), (1, 1), (0, 0))).astype(
        jnp.bfloat16)
    w2 = jnp.transpose(conv_w, (2, 3, 4, 1, 0)).astype(jnp.bfloat16)
    dw2 = jnp.transpose(deconv_w, (2, 3, 4, 0, 1)).astype(jnp.bfloat16)
    dwc = _phase_weights(dw2, CMID, COUT)
    b2 = conv_b.reshape(1, CMID).astype(jnp.float32)
    db2 = jnp.tile(deconv_b.reshape(1, COUT), (1, 8)).astype(jnp.float32)

    body = functools.partial(_fused_kernel, D=D, H=H, W=W, CMID=CMID,
                             COUT=COUT)
    yph = pl.pallas_call(
        body,
        out_shape=jax.ShapeDtypeStruct((N, D, H, W, 8 * COUT), jnp.float32),
        grid=(N,),
        in_specs=[
            pl.BlockSpec((1, D + 2, H + 2, W + 2, CIN),
                         lambda n: (n, 0, 0, 0, 0)),
            pl.BlockSpec((3, 3, 3, CIN, CMID), lambda n: (0, 0, 0, 0, 0)),
            pl.BlockSpec((1, CMID), lambda n: (0, 0)),
            pl.BlockSpec((8, CMID, 8 * COUT), lambda n: (0, 0, 0)),
            pl.BlockSpec((1, 8 * COUT), lambda n: (0, 0)),
        ],
        out_specs=pl.BlockSpec((1, D, H, W, 8 * COUT),
                               lambda n: (n, 0, 0, 0, 0)),
        scratch_shapes=[
            pltpu.VMEM((D + 1, H + 1, W + 1, CMID), jnp.bfloat16),
            pltpu.VMEM((3, 3, (D + 2) * H * W, CIN), jnp.bfloat16),
            pltpu.VMEM((2, 2, (D + 1) * H * W, CMID), jnp.bfloat16),
        ],
        compiler_params=pltpu.CompilerParams(
            dimension_semantics=("parallel",)),
    )(xp, w2, b2, dwc, db2)

    yph = yph.reshape(N, D, H, W, 2, 2, 2, COUT)
    y = jnp.transpose(yph, (0, 7, 1, 4, 2, 5, 3, 6))
    return y.reshape(N, COUT, 2 * D, 2 * H, 2 * W)


# speedup vs baseline: 2.3451x; 2.3134x over previous
import functools

import jax
import jax.numpy as jnp
from jax.experimental import pallas as pl
from jax.experimental.pallas import tpu as pltpu


def _taps(parity):
    return ((0, 1),) if parity == 0 else ((1, 0), (0, 2))


def _phase_weights(dw2, CMID, COUT):
    dwc = jnp.zeros((8, CMID, 8 * COUT), dw2.dtype)
    for od in range(2):
        for oh in range(2):
            for ow in range(2):
                s = od * 4 + oh * 2 + ow
                for pd in range(2):
                    for ph in range(2):
                        for pw in range(2):
                            p = pd * 4 + ph * 2 + pw
                            dims = []
                            ok = True
                            for parity, off in ((pd, od), (ph, oh), (pw, ow)):
                                k = dict(_taps(parity)).get(off)
                                if k is None:
                                    ok = False
                                    break
                                dims.append(k)
                            if ok:
                                kd, kh, kw = dims
                                dwc = dwc.at[s, :, p * COUT:(p + 1) * COUT
                                             ].set(dw2[kd, kh, kw])
    return dwc


def _fused_kernel(x_ref, w_ref, b_ref, dwc_ref, db_ref, o_ref,
                  h_ref, xs_ref, hs_ref, *, D, H, W, CMID, COUT):
    cin = x_ref.shape[-1]
    R = H * W
    if True:
        for b in range(3):
            for c in range(3):
                xs_ref[b, c] = x_ref[0, :, b:b + H, c:c + W, :].reshape(
                    (D + 2) * R, cin)

        h_ref[...] = jnp.zeros_like(h_ref)
        DC = 8 if D % 8 == 0 else D
        for d0 in range(0, D, DC):
            M = DC * R
            acc = jnp.zeros((M, CMID), jnp.float32)
            for a in range(3):
                for b in range(3):
                    for c in range(3):
                        tap = xs_ref[b, c, (d0 + a) * R:(d0 + a + DC) * R, :]
                        acc = acc + jnp.dot(
                            tap, w_ref[a, b, c],
                            preferred_element_type=jnp.float32)
            acc = jnp.maximum(acc + b_ref[...], 0.0)
            h_ref[d0:d0 + DC, :H, :W, :] = acc.reshape(
                DC, H, W, CMID).astype(jnp.bfloat16)

        for oh in range(2):
            for ow in range(2):
                hs_ref[oh, ow] = h_ref[:, oh:oh + H, ow:ow + W, :].reshape(
                    (D + 1) * R, CMID)

    DCH = 2
    for mc in range(D // DCH):
        M = DCH * R
        acc = jnp.zeros((M, 8 * COUT), jnp.float32)
        for od in range(2):
            for oh in range(2):
                for ow in range(2):
                    s = od * 4 + oh * 2 + ow
                    d0 = (mc * DCH + od) * R
                    tap = hs_ref[oh, ow, d0:d0 + M, :]
                    acc = acc + jnp.dot(
                        tap, dwc_ref[s],
                        preferred_element_type=jnp.float32)
        acc = jnp.maximum(acc + db_ref[...], 0.0)
        for pd in range(2):
            for ph in range(2):
                cols = (pd * 2 + ph) * 2 * COUT
                o_ref[0, mc * DCH:(mc + 1) * DCH, pd, :, ph] = acc[
                    :, cols:cols + 2 * COUT].reshape(DCH, H, W, 2, COUT)


def kernel(conv_w, conv_b, deconv_w, deconv_b, x_ncdhw):
    N, CIN, D, H, W = x_ncdhw.shape
    CMID = conv_w.shape[0]
    COUT = deconv_w.shape[1]

    x = jnp.transpose(x_ncdhw, (0, 2, 3, 4, 1))
    xp = jnp.pad(x, ((0, 0), (1, 1), (1, 1), (1, 1), (0, 0))).astype(
        jnp.bfloat16)
    w2 = jnp.transpose(conv_w, (2, 3, 4, 1, 0)).astype(jnp.bfloat16)
    dw2 = jnp.transpose(deconv_w, (2, 3, 4, 0, 1)).astype(jnp.bfloat16)
    dwc = _phase_weights(dw2, CMID, COUT)
    b2 = conv_b.reshape(1, CMID).astype(jnp.float32)
    db2 = jnp.tile(deconv_b.reshape(1, COUT), (1, 8)).astype(jnp.float32)

    body = functools.partial(_fused_kernel, D=D, H=H, W=W, CMID=CMID,
                             COUT=COUT)
    yph = pl.pallas_call(
        body,
        out_shape=jax.ShapeDtypeStruct((N, D, 2, H, 2, W, 2, COUT),
                                       jnp.float32),
        grid=(N,),
        in_specs=[
            pl.BlockSpec((1, D + 2, H + 2, W + 2, CIN),
                         lambda n: (n, 0, 0, 0, 0)),
            pl.BlockSpec((3, 3, 3, CIN, CMID), lambda n: (0, 0, 0, 0, 0)),
            pl.BlockSpec((1, CMID), lambda n: (0, 0)),
            pl.BlockSpec((8, CMID, 8 * COUT), lambda n: (0, 0, 0)),
            pl.BlockSpec((1, 8 * COUT), lambda n: (0, 0)),
        ],
        out_specs=pl.BlockSpec((1, D, 2, H, 2, W, 2, COUT),
                               lambda n: (n, 0, 0, 0, 0, 0, 0, 0)),
        scratch_shapes=[
            pltpu.VMEM((D + 1, H + 1, W + 1, CMID), jnp.bfloat16),
            pltpu.VMEM((3, 3, (D + 2) * H * W, CIN), jnp.bfloat16),
            pltpu.VMEM((2, 2, (D + 1) * H * W, CMID), jnp.bfloat16),
        ],
        compiler_params=pltpu.CompilerParams(
            dimension_semantics=("parallel",)),
    )(xp, w2, b2, dwc, db2)

    y = jnp.transpose(yph, (0, 7, 1, 2, 3, 4, 5, 6))
    return y.reshape(N, COUT, 2 * D, 2 * H, 2 * W)
